# Initial kernel scaffold; baseline (speedup 1.0000x reference)
import jax
import jax.numpy as jnp
from jax.experimental import pallas as pl


def kernel(x, batch, Wg, bg, size):
    raise NotImplementedError("write your pallas kernel here")


# speedup vs baseline: 1.8850x; 1.8850x over previous
import jax
import jax.numpy as jnp
from jax.experimental import pallas as pl

_S = 1024
_R = 400


def _k_gate_max(x_ref, b_ref, wg_ref, bg_ref, gate_ref, smax_ref):
    i = pl.program_id(0)
    g = jnp.dot(x_ref[...], wg_ref[...],
                preferred_element_type=jnp.float32) + bg_ref[0, 0]
    gate_ref[...] = g
    iota = jax.lax.broadcasted_iota(jnp.int32, (_R, _S), 1)
    cond = b_ref[...] == iota
    vals = jnp.where(cond, g, -jnp.inf)
    partial = jnp.max(vals, axis=0, keepdims=True)

    @pl.when(i == 0)
    def _():
        smax_ref[...] = jnp.full((1, _S), -jnp.inf, jnp.float32)

    smax_ref[...] = jnp.maximum(smax_ref[...], partial)


def _k_exp_sum(g_ref, b_ref, smax_ref, e_ref, ssum_ref):
    i = pl.program_id(0)
    sm = smax_ref[...]
    smf = jnp.where(jnp.isfinite(sm), sm, 0.0)
    iota = jax.lax.broadcasted_iota(jnp.int32, (_R, _S), 1)
    cond = b_ref[...] == iota
    gm = jnp.max(jnp.where(cond, smf, -jnp.inf), axis=1, keepdims=True)
    e = jnp.exp(g_ref[...] - gm)
    e_ref[...] = e
    partial = jnp.sum(jnp.where(cond, e, 0.0), axis=0, keepdims=True)

    @pl.when(i == 0)
    def _():
        ssum_ref[...] = jnp.zeros((1, _S), jnp.float32)

    ssum_ref[...] += partial


def _k_norm_out(x_ref, e_ref, b_ref, ssum_ref, out_ref, w_ref):
    i = pl.program_id(0)
    ss = ssum_ref[...]
    iota = jax.lax.broadcasted_iota(jnp.int32, (_R, _S), 1)
    cond = b_ref[...] == iota
    gs = jnp.sum(jnp.where(cond, ss, 0.0), axis=1, keepdims=True)
    w = e_ref[...] / (gs + 1e-16)
    w_ref[...] = w
    wx = w * x_ref[...]
    onehot = cond.astype(jnp.float32)
    partial = jax.lax.dot_general(onehot, wx, (((0,), (0,)), ((), ())),
                                  preferred_element_type=jnp.float32)

    @pl.when(i == 0)
    def _():
        out_ref[...] = jnp.zeros_like(out_ref)

    out_ref[...] += partial


def kernel(x, batch, Wg, bg, size):
    n, d = x.shape
    del size
    grid = n // _R
    b2 = batch.reshape(n, 1).astype(jnp.int32)
    bg2 = bg.reshape(1, 1).astype(jnp.float32)

    row = lambda i: (i, 0)
    const = lambda i: (0, 0)
    f32 = jnp.float32

    gate, smax = pl.pallas_call(
        _k_gate_max,
        grid=(grid,),
        in_specs=[pl.BlockSpec((_R, d), row),
                  pl.BlockSpec((_R, 1), row),
                  pl.BlockSpec((d, 1), const),
                  pl.BlockSpec((1, 1), const)],
        out_specs=[pl.BlockSpec((_R, 1), row),
                   pl.BlockSpec((1, _S), const)],
        out_shape=[jax.ShapeDtypeStruct((n, 1), f32),
                   jax.ShapeDtypeStruct((1, _S), f32)],
    )(x, b2, Wg, bg2)

    e, ssum = pl.pallas_call(
        _k_exp_sum,
        grid=(grid,),
        in_specs=[pl.BlockSpec((_R, 1), row),
                  pl.BlockSpec((_R, 1), row),
                  pl.BlockSpec((1, _S), const)],
        out_specs=[pl.BlockSpec((_R, 1), row),
                   pl.BlockSpec((1, _S), const)],
        out_shape=[jax.ShapeDtypeStruct((n, 1), f32),
                   jax.ShapeDtypeStruct((1, _S), f32)],
    )(gate, b2, smax)

    out, w = pl.pallas_call(
        _k_norm_out,
        grid=(grid,),
        in_specs=[pl.BlockSpec((_R, d), row),
                  pl.BlockSpec((_R, 1), row),
                  pl.BlockSpec((_R, 1), row),
                  pl.BlockSpec((1, _S), const)],
        out_specs=[pl.BlockSpec((_S, d), const),
                   pl.BlockSpec((_R, 1), row)],
        out_shape=[jax.ShapeDtypeStruct((_S, d), f32),
                   jax.ShapeDtypeStruct((n, 1), f32)],
    )(x, e, b2, ssum)

    return (out, w)


# speedup vs baseline: 3.4596x; 1.8354x over previous
import jax
import jax.numpy as jnp
from jax import lax
from jax.experimental import pallas as pl
from jax.experimental.pallas import tpu as pltpu
from jax.experimental.pallas import tpu_sc as plsc

_S = 1024
_D = 128
_R = 400
_NW = 32
_GPW = 196
_NG = 6250
_SB = 1088
_TB = 1152
_AR = 2176


def _k_gate_max(x_ref, b_ref, wg_ref, bg_ref, gate_ref, smax_ref):
    i = pl.program_id(0)
    g = jnp.dot(x_ref[...], wg_ref[...],
                preferred_element_type=jnp.float32) + bg_ref[0, 0]
    gate_ref[...] = g
    iota = jax.lax.broadcasted_iota(jnp.int32, (_R, _S), 1)
    cond = b_ref[...] == iota
    vals = jnp.where(cond, g, -jnp.inf)
    partial = jnp.max(vals, axis=0, keepdims=True)

    @pl.when(i == 0)
    def _():
        smax_ref[...] = jnp.full((1, _S), -jnp.inf, jnp.float32)

    smax_ref[...] = jnp.maximum(smax_ref[...], partial)


def _stage_table(hbm, buf):
    pltpu.sync_copy(hbm, buf.at[pl.ds(0, _S)])
    for i in range(8):
        buf[pl.ds(_S + i * 16, 16)] = jnp.zeros((16,), jnp.float32)

    def san(i, _):
        v = buf[pl.ds(i * 16, 16)]
        buf[pl.ds(i * 16, 16)] = jnp.where(v > -jnp.inf, v, 0.0)
        return 0

    lax.fori_loop(0, _S // 16, san, 0)


def _sc_accumulate(x_hbm, gate_hbm, b_hbm, smax_hbm, part_hbm,
                   batch_buf, gate_buf, smax_buf, xb, flushbuf,
                   idxbuf, zbuf, acc_sh, sem0, sem1):
    cid = lax.axis_index("c")
    sid = lax.axis_index("s")
    wid = cid * 16 + sid
    ng = jnp.minimum(_GPW, _NG - wid * _GPW)
    g0 = wid * _GPW
    row0 = g0 * 16

    def zb(r, _):
        for k in range(8):
            zbuf[r, pl.ds(k * 16, 16)] = jnp.zeros((16,), jnp.float32)
        return 0
    lax.fori_loop(0, 136, zb, 0)
    pltpu.sync_copy(zbuf, acc_sh.at[pl.ds(sid * 136, 136)])
    def fb(r, _):
        for k in range(8):
            flushbuf[r, pl.ds(k * 16, 16)] = jnp.zeros((16,), jnp.float32)
        return 0
    lax.fori_loop(0, 16, fb, 0)
    plsc.subcore_barrier()

    pltpu.sync_copy(b_hbm.at[pl.ds(row0, _GPW * 16)], batch_buf)
    pltpu.sync_copy(gate_hbm.at[pl.ds(row0, _GPW * 16)], gate_buf)
    _stage_table(smax_hbm, smax_buf)

    iota16 = lax.iota(jnp.int32, 16)
    sems = (sem0, sem1)

    def xsrc(g):
        gg = jnp.minimum(g, ng - 1)
        return x_hbm.at[pl.ds((g0 + gg) * 16, 16), :]

    for q in (0, 1):
        pltpu.async_copy(xsrc(jnp.int32(q)), xb.at[q], sems[q])

    def dma_flush(pseg):
        idxbuf[...] = pseg + jnp.where(iota16 == 1, _SB, 0)
        pltpu.sync_copy(flushbuf, acc_sh.at[idxbuf], add=True)
        return jnp.int32(0)

    def no_dma(pseg):
        return jnp.int32(0)

    zeros9 = tuple(jnp.zeros((16,), jnp.float32) for _ in range(9))

    def pair_body(p, carry):
        for q in (0, 1):
            g = 2 * p + q
            pltpu.make_async_copy(x_hbm.at[pl.ds(0, 16), :],
                                  xb.at[q], sems[q]).wait()
            l = g * 16
            bvec = batch_buf[pl.ds(l, 16)]
            gvec = gate_buf[pl.ds(l, 16)]
            for r in range(16):
                seg = bvec[r]
                smrow = smax_buf[pl.ds(seg, 16)]
                e_r = jnp.exp(gvec[r] - smrow)[0]
                prev, vecs = carry
                pred = (seg != prev) & (prev >= 0)
                lax.cond(pred, dma_flush, no_dma, prev)
                keep = jnp.where(pred, 0.0, 1.0)
                new = [vecs[k] * keep + e_r * xb[q, r, pl.ds(k * 16, 16)]
                       for k in range(8)]
                new.append(vecs[8] * keep + e_r)
                for k in range(8):
                    flushbuf[0, pl.ds(k * 16, 16)] = new[k]
                flushbuf[1, pl.ds(0, 16)] = jnp.where(iota16 == 0, new[8], 0.0)
                carry = (seg, tuple(new))
            pltpu.async_copy(xsrc(g + 2), xb.at[q], sems[q])
        return carry

    carry = lax.fori_loop(0, _GPW // 2, pair_body,
                          (jnp.int32(-1), zeros9))
    for q in (0, 1):
        pltpu.make_async_copy(x_hbm.at[pl.ds(0, 16), :],
                              xb.at[q], sems[q]).wait()

    prev, vecs = carry
    lax.cond(prev >= 0, dma_flush, no_dma, prev)

    plsc.subcore_barrier()
    pltpu.sync_copy(acc_sh.at[pl.ds(sid * 136, 136)],
                    part_hbm.at[cid, pl.ds(sid * 136, 136), :])


def _k_combine(p_ref, out_ref, ssum_ref):
    p = p_ref[...]
    acc = p[0, :_S, :] + p[1, :_S, :]
    s = p[0, _SB:_SB + _S, :1] + p[1, _SB:_SB + _S, :1]
    ssum_ref[...] = s
    out_ref[...] = acc / (s + 1e-16)


def _sc_wpass(gate_hbm, b_hbm, smax_hbm, ssum_hbm, w_hbm,
              batch_buf, gate_buf, wbuf, smax_buf, ssum_buf):
    cid = lax.axis_index("c")
    sid = lax.axis_index("s")
    wid = cid * 16 + sid
    row0 = wid * _GPW * 16

    pltpu.sync_copy(b_hbm.at[pl.ds(row0, _GPW * 16)], batch_buf)
    pltpu.sync_copy(gate_hbm.at[pl.ds(row0, _GPW * 16)], gate_buf)
    _stage_table(smax_hbm, smax_buf)
    pltpu.sync_copy(ssum_hbm, ssum_buf.at[pl.ds(0, _S)])
    for i in range(8):
        ssum_buf[pl.ds(_S + i * 16, 16)] = jnp.zeros((16,), jnp.float32)

    iota16 = lax.iota(jnp.int32, 16)

    def body(g, _):
        l = g * 16
        bvec = batch_buf[pl.ds(l, 16)]
        gvec = gate_buf[pl.ds(l, 16)]
        wacc = jnp.zeros((16,), jnp.float32)
        for r in range(16):
            seg = bvec[r]
            smrow = smax_buf[pl.ds(seg, 16)]
            ssrow = ssum_buf[pl.ds(seg, 16)]
            wrv = jnp.exp(gvec[r] - smrow) / (ssrow + 1e-16)
            wacc = jnp.where(iota16 == r, wrv[0], wacc)
        wbuf[pl.ds(l, 16)] = wacc
        return 0

    lax.fori_loop(0, _GPW, body, 0)
    pltpu.sync_copy(wbuf, w_hbm.at[pl.ds(row0, _GPW * 16)])


def kernel(x, batch, Wg, bg, size):
    n, d = x.shape
    del size
    grid = n // _R
    npad = _NW * _GPW * 16
    b2 = batch.reshape(n, 1).astype(jnp.int32)
    bg2 = bg.reshape(1, 1).astype(jnp.float32)

    row = lambda i: (i, 0)
    const = lambda i: (0, 0)
    f32 = jnp.float32

    gate, smax = pl.pallas_call(
        _k_gate_max,
        grid=(grid,),
        in_specs=[pl.BlockSpec((_R, d), row),
                  pl.BlockSpec((_R, 1), row),
                  pl.BlockSpec((d, 1), const),
                  pl.BlockSpec((1, 1), const)],
        out_specs=[pl.BlockSpec((_R, 1), row),
                   pl.BlockSpec((1, _S), const)],
        out_shape=[jax.ShapeDtypeStruct((n, 1), f32),
                   jax.ShapeDtypeStruct((1, _S), f32)],
    )(x, b2, Wg, bg2)

    gate_flat = jnp.pad(gate.reshape(-1), (0, npad - n))
    b_flat = jnp.pad(batch.astype(jnp.int32), (0, npad - n),
                     constant_values=_S)
    smax_flat = smax.reshape(-1)

    mesh = plsc.VectorSubcoreMesh(core_axis_name="c", subcore_axis_name="s")
    chunk = _GPW * 16

    part = pl.kernel(
        _sc_accumulate,
        out_type=jax.ShapeDtypeStruct((2, _AR, _D), f32),
        mesh=mesh,
        scratch_types=[
            pltpu.VMEM((chunk,), jnp.int32),
            pltpu.VMEM((chunk,), f32),
            pltpu.VMEM((_TB,), f32),
            pltpu.VMEM((2, 16, _D), f32),
            pltpu.VMEM((16, _D), f32),
            pltpu.VMEM((16,), jnp.int32),
            pltpu.VMEM((136, _D), f32),
            pltpu.VMEM_SHARED((_AR, _D), f32),
            pltpu.SemaphoreType.DMA,
            pltpu.SemaphoreType.DMA,
        ],
    )(x, gate_flat, b_flat, smax_flat)

    out, ssum = pl.pallas_call(
        _k_combine,
        grid=(1,),
        in_specs=[pl.BlockSpec((2, _AR, _D), lambda i: (0, 0, 0))],
        out_specs=[pl.BlockSpec((_S, _D), lambda i: (0, 0)),
                   pl.BlockSpec((_S, 1), lambda i: (0, 0))],
        out_shape=[jax.ShapeDtypeStruct((_S, _D), f32),
                   jax.ShapeDtypeStruct((_S, 1), f32)],
    )(part)

    w_flat = pl.kernel(
        _sc_wpass,
        out_type=jax.ShapeDtypeStruct((npad,), f32),
        mesh=mesh,
        scratch_types=[
            pltpu.VMEM((chunk,), jnp.int32),
            pltpu.VMEM((chunk,), f32),
            pltpu.VMEM((chunk,), f32),
            pltpu.VMEM((_TB,), f32),
            pltpu.VMEM((_TB,), f32),
        ],
    )(gate_flat, b_flat, smax_flat, ssum.reshape(-1))

    return (out, w_flat[:n].reshape(n, 1))


# speedup vs baseline: 3.9893x; 1.1531x over previous
import jax
import jax.numpy as jnp
from jax import lax
from jax.experimental import pallas as pl
from jax.experimental.pallas import tpu as pltpu
from jax.experimental.pallas import tpu_sc as plsc

_S = 1024
_D = 128
_R = 400
_NW = 32
_GPW = 196
_NG = 6250
_SB = 1088
_TB = 1152
_AR = 2176


def _k_gate_max(x_ref, b_ref, wg_ref, bg_ref, gate_ref, smax_ref):
    i = pl.program_id(0)
    g = jnp.dot(x_ref[...], wg_ref[...],
                preferred_element_type=jnp.float32) + bg_ref[0, 0]
    gate_ref[...] = g.reshape(1, 1, _R)
    iota = jax.lax.broadcasted_iota(jnp.int32, (_R, _S), 1)
    cond = b_ref[...].reshape(_R, 1) == iota
    vals = jnp.where(cond, g, -jnp.inf)
    partial = jnp.max(vals, axis=0, keepdims=True)

    @pl.when(i == 0)
    def _():
        smax_ref[...] = jnp.full((1, _S), -jnp.inf, jnp.float32)

    smax_ref[...] = jnp.maximum(smax_ref[...], partial)


def _stage_table(hbm, buf):
    pltpu.sync_copy(hbm, buf.at[pl.ds(0, _S)])
    for i in range(8):
        buf[pl.ds(_S + i * 16, 16)] = jnp.zeros((16,), jnp.float32)

    def san(i, _):
        v = buf[pl.ds(i * 16, 16)]
        buf[pl.ds(i * 16, 16)] = jnp.where(v > -jnp.inf, v, 0.0)
        return 0

    lax.fori_loop(0, _S // 16, san, 0)


def _sc_accumulate(x_hbm, gate_hbm, b_hbm, smax_hbm, part_hbm,
                   batch_buf, gate_buf, smax_buf, xb, flushbuf,
                   idxbuf, zbuf, acc_sh, sem0, sem1):
    cid = lax.axis_index("c")
    sid = lax.axis_index("s")
    wid = cid * 16 + sid
    ng = jnp.minimum(_GPW, _NG - wid * _GPW)
    g0 = wid * _GPW
    row0 = g0 * 16

    def zb(r, _):
        for k in range(8):
            zbuf[r, pl.ds(k * 16, 16)] = jnp.zeros((16,), jnp.float32)
        return 0
    lax.fori_loop(0, 136, zb, 0)
    pltpu.sync_copy(zbuf, acc_sh.at[pl.ds(sid * 136, 136)])
    def fb(r, _):
        for k in range(8):
            flushbuf[r, pl.ds(k * 16, 16)] = jnp.zeros((16,), jnp.float32)
        return 0
    lax.fori_loop(0, 16, fb, 0)
    plsc.subcore_barrier()

    pltpu.sync_copy(b_hbm.at[pl.ds(row0, _GPW * 16)], batch_buf)
    pltpu.sync_copy(gate_hbm.at[pl.ds(row0, _GPW * 16)], gate_buf)
    _stage_table(smax_hbm, smax_buf)

    iota16 = lax.iota(jnp.int32, 16)
    sems = (sem0, sem1)

    def xsrc(g):
        gg = jnp.minimum(g, ng - 1)
        return x_hbm.at[pl.ds((g0 + gg) * 16, 16), :]

    for q in (0, 1):
        pltpu.async_copy(xsrc(jnp.int32(q)), xb.at[q], sems[q])

    def dma_flush(pseg):
        idxbuf[...] = pseg + jnp.where(iota16 == 1, _SB, 0)
        pltpu.sync_copy(flushbuf, acc_sh.at[idxbuf], add=True)
        return jnp.int32(0)

    def no_dma(pseg):
        return jnp.int32(0)

    zeros9 = tuple(jnp.zeros((16,), jnp.float32) for _ in range(9))

    def pair_body(p, carry):
        for q in (0, 1):
            g = 2 * p + q
            pltpu.make_async_copy(x_hbm.at[pl.ds(0, 16), :],
                                  xb.at[q], sems[q]).wait()
            l = g * 16
            bvec = batch_buf[pl.ds(l, 16)]
            gvec = gate_buf[pl.ds(l, 16)]
            for r in range(16):
                seg = bvec[r]
                smrow = smax_buf[pl.ds(seg, 16)]
                e_r = jnp.exp(gvec[r] - smrow)[0]
                prev, vecs = carry
                pred = (seg != prev) & (prev >= 0)
                lax.cond(pred, dma_flush, no_dma, prev)
                keep = jnp.where(pred, 0.0, 1.0)
                new = [vecs[k] * keep + e_r * xb[q, r, pl.ds(k * 16, 16)]
                       for k in range(8)]
                new.append(vecs[8] * keep + e_r)
                for k in range(8):
                    flushbuf[0, pl.ds(k * 16, 16)] = new[k]
                flushbuf[1, pl.ds(0, 16)] = jnp.where(iota16 == 0, new[8], 0.0)
                carry = (seg, tuple(new))
            pltpu.async_copy(xsrc(g + 2), xb.at[q], sems[q])
        return carry

    carry = lax.fori_loop(0, _GPW // 2, pair_body,
                          (jnp.int32(-1), zeros9))
    for q in (0, 1):
        pltpu.make_async_copy(x_hbm.at[pl.ds(0, 16), :],
                              xb.at[q], sems[q]).wait()

    prev, vecs = carry
    lax.cond(prev >= 0, dma_flush, no_dma, prev)

    plsc.subcore_barrier()
    pltpu.sync_copy(acc_sh.at[pl.ds(sid * 136, 136)],
                    part_hbm.at[cid, pl.ds(sid * 136, 136), :])


def _k_combine(p_ref, out_ref, ssum_ref):
    p = p_ref[...]
    acc = p[0, :_S, :] + p[1, :_S, :]
    s = p[0, _SB:_SB + _S, :1] + p[1, _SB:_SB + _S, :1]
    ssum_ref[...] = s
    out_ref[...] = acc / (s + 1e-16)


def _sc_wpass(gate_hbm, b_hbm, smax_hbm, ssum_hbm, w_hbm,
              batch_buf, gate_buf, wbuf, smax_buf, ssum_buf):
    cid = lax.axis_index("c")
    sid = lax.axis_index("s")
    wid = cid * 16 + sid
    row0 = wid * _GPW * 16

    pltpu.sync_copy(b_hbm.at[pl.ds(row0, _GPW * 16)], batch_buf)
    pltpu.sync_copy(gate_hbm.at[pl.ds(row0, _GPW * 16)], gate_buf)
    _stage_table(smax_hbm, smax_buf)
    pltpu.sync_copy(ssum_hbm, ssum_buf.at[pl.ds(0, _S)])
    for i in range(8):
        ssum_buf[pl.ds(_S + i * 16, 16)] = jnp.zeros((16,), jnp.float32)

    iota16 = lax.iota(jnp.int32, 16)

    def body(g, _):
        l = g * 16
        bvec = batch_buf[pl.ds(l, 16)]
        gvec = gate_buf[pl.ds(l, 16)]
        wacc = jnp.zeros((16,), jnp.float32)
        for r in range(16):
            seg = bvec[r]
            smrow = smax_buf[pl.ds(seg, 16)]
            ssrow = ssum_buf[pl.ds(seg, 16)]
            wrv = jnp.exp(gvec[r] - smrow) / (ssrow + 1e-16)
            wacc = jnp.where(iota16 == r, wrv[0], wacc)
        wbuf[pl.ds(l, 16)] = wacc
        return 0

    lax.fori_loop(0, _GPW, body, 0)
    pltpu.sync_copy(wbuf, w_hbm.at[pl.ds(row0, _GPW * 16)])


def kernel(x, batch, Wg, bg, size):
    n, d = x.shape
    del size
    grid = n // _R
    npad = _NW * _GPW * 16
    b2 = batch.astype(jnp.int32).reshape(grid, 1, _R)
    bg2 = bg.reshape(1, 1).astype(jnp.float32)

    row = lambda i: (i, 0)
    row3 = lambda i: (i, 0, 0)
    const = lambda i: (0, 0)
    f32 = jnp.float32

    gate, smax = pl.pallas_call(
        _k_gate_max,
        grid=(grid,),
        in_specs=[pl.BlockSpec((_R, d), row),
                  pl.BlockSpec((1, 1, _R), row3),
                  pl.BlockSpec((d, 1), const),
                  pl.BlockSpec((1, 1), const)],
        out_specs=[pl.BlockSpec((1, 1, _R), row3),
                   pl.BlockSpec((1, _S), const)],
        out_shape=[jax.ShapeDtypeStruct((grid, 1, _R), f32),
                   jax.ShapeDtypeStruct((1, _S), f32)],
    )(x, b2, Wg, bg2)

    gate_flat = jnp.pad(gate.reshape(-1), (0, npad - n))
    b_flat = jnp.pad(batch.astype(jnp.int32), (0, npad - n),
                     constant_values=_S)
    smax_flat = smax.reshape(-1)

    mesh = plsc.VectorSubcoreMesh(core_axis_name="c", subcore_axis_name="s")
    chunk = _GPW * 16

    part = pl.kernel(
        _sc_accumulate,
        out_type=jax.ShapeDtypeStruct((2, _AR, _D), f32),
        mesh=mesh,
        scratch_types=[
            pltpu.VMEM((chunk,), jnp.int32),
            pltpu.VMEM((chunk,), f32),
            pltpu.VMEM((_TB,), f32),
            pltpu.VMEM((2, 16, _D), f32),
            pltpu.VMEM((16, _D), f32),
            pltpu.VMEM((16,), jnp.int32),
            pltpu.VMEM((136, _D), f32),
            pltpu.VMEM_SHARED((_AR, _D), f32),
            pltpu.SemaphoreType.DMA,
            pltpu.SemaphoreType.DMA,
        ],
    )(x, gate_flat, b_flat, smax_flat)

    out, ssum = pl.pallas_call(
        _k_combine,
        grid=(1,),
        in_specs=[pl.BlockSpec((2, _AR, _D), lambda i: (0, 0, 0))],
        out_specs=[pl.BlockSpec((_S, _D), lambda i: (0, 0)),
                   pl.BlockSpec((_S, 1), lambda i: (0, 0))],
        out_shape=[jax.ShapeDtypeStruct((_S, _D), f32),
                   jax.ShapeDtypeStruct((_S, 1), f32)],
    )(part)

    w_flat = pl.kernel(
        _sc_wpass,
        out_type=jax.ShapeDtypeStruct((npad,), f32),
        mesh=mesh,
        scratch_types=[
            pltpu.VMEM((chunk,), jnp.int32),
            pltpu.VMEM((chunk,), f32),
            pltpu.VMEM((chunk,), f32),
            pltpu.VMEM((_TB,), f32),
            pltpu.VMEM((_TB,), f32),
        ],
    )(gate_flat, b_flat, smax_flat, ssum.reshape(-1))

    return (out, w_flat[:n].reshape(n, 1))
